# baseline (speedup 1.0000x reference)
import functools

import jax
import jax.numpy as jnp
from jax import lax
from jax.experimental import pallas as pl
import jax.experimental.pallas.tpu as pltpu
from jax.experimental.pallas import tpu_sc as plsc

_BATCH = 4096
_N = 100000
_D = 128
_TEMP = 0.05
_NT = 512
_LOG2E = 1.4426950408889634

_SC_NC = 2
_SC_NS = 16
_SC_NW = _SC_NC * _SC_NS
_B_PER_W = _BATCH // _SC_NW


def _norm_kernel(x_ref, xs_ref):
    x = x_ref[...]
    nrm = jnp.sqrt(jnp.sum(x * x, axis=1, keepdims=True))
    scale = _LOG2E / (jnp.maximum(nrm, 1e-12) * _TEMP)
    xs_ref[...] = (x * scale).astype(jnp.bfloat16)


def _sum_kernel(xs_ref, f_ref, s_ref):
    i = pl.program_id(0)

    @pl.when(i == 0)
    def _init():
        s_ref[...] = jnp.zeros_like(s_ref)

    l2 = lax.dot_general(
        xs_ref[...], f_ref[...],
        (((1,), (1,)), ((), ())),
        preferred_element_type=jnp.float32,
    )
    e = jnp.exp2(l2.astype(jnp.bfloat16))
    e128 = (e[:, 0:128] + e[:, 128:256]) + (e[:, 256:384] + e[:, 384:512])
    s_ref[...] += e128.astype(jnp.float32)


def _sc_gather_body(table_hbm, idx_hbm, out_hbm, idx_v, rows_v, sem):
    wid = lax.axis_index("s") * _SC_NC + lax.axis_index("c")
    base = wid * _B_PER_W
    pltpu.sync_copy(idx_hbm.at[pl.ds(base, _B_PER_W)], idx_v)
    pltpu.async_copy(table_hbm.at[idx_v], rows_v, sem).wait()
    pltpu.sync_copy(rows_v, out_hbm.at[pl.ds(base, _B_PER_W)])


def _fin_kernel(x_ref, g_ref, s_ref, out_ref):
    x = x_ref[...]
    nrm = jnp.sqrt(jnp.sum(x * x, axis=1, keepdims=True))
    t = jnp.sum(x * g_ref[...], axis=1, keepdims=True) / (
        jnp.maximum(nrm, 1e-12) * _TEMP)
    s = jnp.sum(s_ref[...], axis=1, keepdims=True)
    out_ref[...] = jnp.sum(jnp.log(s) - t, axis=0, keepdims=True)


@functools.partial(jax.jit, static_argnames=())
def kernel(inputs, targets, features):
    n_pad = pl.cdiv(_N, _NT) * _NT
    f = jnp.pad(features, ((0, n_pad - _N), (0, 0))).astype(jnp.bfloat16)
    tgt = targets.astype(jnp.int32)

    xs = pl.pallas_call(
        _norm_kernel,
        out_shape=jax.ShapeDtypeStruct((_BATCH, _D), jnp.bfloat16),
    )(inputs)

    s128 = pl.pallas_call(
        _sum_kernel,
        grid=(n_pad // _NT,),
        in_specs=[
            pl.BlockSpec((_BATCH, _D), lambda i: (0, 0)),
            pl.BlockSpec((_NT, _D), lambda i: (i, 0)),
        ],
        out_specs=pl.BlockSpec((_BATCH, _D), lambda i: (0, 0)),
        out_shape=jax.ShapeDtypeStruct((_BATCH, _D), jnp.float32),
    )(xs, f)

    gathered = pl.kernel(
        _sc_gather_body,
        out_type=jax.ShapeDtypeStruct((_BATCH, _D), jnp.float32),
        mesh=plsc.VectorSubcoreMesh(core_axis_name="c", subcore_axis_name="s"),
        scratch_types=[
            pltpu.VMEM((_B_PER_W,), jnp.int32),
            pltpu.VMEM((_B_PER_W, _D), jnp.float32),
            pltpu.SemaphoreType.DMA,
        ],
    )(features, tgt)

    out = pl.pallas_call(
        _fin_kernel,
        out_shape=jax.ShapeDtypeStruct((1, 1), jnp.float32),
    )(inputs, gathered, s128)
    return out[0, 0] / _BATCH


# speedup vs baseline: 1.0918x; 1.0918x over previous
import functools

import jax
import jax.numpy as jnp
from jax import lax
from jax.experimental import pallas as pl
import jax.experimental.pallas.tpu as pltpu
from jax.experimental.pallas import tpu_sc as plsc

_BATCH = 4096
_N = 100000
_D = 128
_TEMP = 0.05
_NT = 512
_LOG2E = 1.4426950408889634

_SC_NC = 2
_SC_NS = 16
_SC_NW = _SC_NC * _SC_NS
_B_PER_W = _BATCH // _SC_NW


def _norm_kernel(x_ref, xs_ref):
    x = x_ref[...]
    nrm = jnp.sqrt(jnp.sum(x * x, axis=1, keepdims=True))
    scale = _LOG2E / (jnp.maximum(nrm, 1e-12) * _TEMP)
    xs_ref[...] = (x * scale).astype(jnp.float8_e4m3fn)


def _sum_kernel(xs_ref, f_ref, s_ref):
    i = pl.program_id(0)

    @pl.when(i == 0)
    def _init():
        s_ref[...] = jnp.zeros_like(s_ref)

    l2 = lax.dot_general(
        xs_ref[...], f_ref[...],
        (((1,), (1,)), ((), ())),
        preferred_element_type=jnp.float32,
    )
    e = jnp.exp2(l2.astype(jnp.bfloat16))
    e128 = (e[:, 0:128] + e[:, 128:256]) + (e[:, 256:384] + e[:, 384:512])
    s_ref[...] += e128.astype(jnp.float32)


def _sc_gather_body(table_hbm, idx_hbm, out_hbm, idx_v, rows_v, sem):
    wid = lax.axis_index("s") * _SC_NC + lax.axis_index("c")
    base = wid * _B_PER_W
    pltpu.sync_copy(idx_hbm.at[pl.ds(base, _B_PER_W)], idx_v)
    pltpu.async_copy(table_hbm.at[idx_v], rows_v, sem).wait()
    pltpu.sync_copy(rows_v, out_hbm.at[pl.ds(base, _B_PER_W)])


def _fin_kernel(x_ref, g_ref, s_ref, out_ref):
    x = x_ref[...]
    nrm = jnp.sqrt(jnp.sum(x * x, axis=1, keepdims=True))
    t = jnp.sum(x * g_ref[...], axis=1, keepdims=True) / (
        jnp.maximum(nrm, 1e-12) * _TEMP)
    s = jnp.sum(s_ref[...], axis=1, keepdims=True)
    out_ref[...] = jnp.sum(jnp.log(s) - t, axis=0, keepdims=True)


@functools.partial(jax.jit, static_argnames=())
def kernel(inputs, targets, features):
    n_pad = pl.cdiv(_N, _NT) * _NT
    f = jnp.pad(features, ((0, n_pad - _N), (0, 0))).astype(jnp.float8_e4m3fn)
    tgt = targets.astype(jnp.int32)

    xs = pl.pallas_call(
        _norm_kernel,
        out_shape=jax.ShapeDtypeStruct((_BATCH, _D), jnp.float8_e4m3fn),
    )(inputs)

    s128 = pl.pallas_call(
        _sum_kernel,
        grid=(n_pad // _NT,),
        in_specs=[
            pl.BlockSpec((_BATCH, _D), lambda i: (0, 0)),
            pl.BlockSpec((_NT, _D), lambda i: (i, 0)),
        ],
        out_specs=pl.BlockSpec((_BATCH, _D), lambda i: (0, 0)),
        out_shape=jax.ShapeDtypeStruct((_BATCH, _D), jnp.float32),
    )(xs, f)

    gathered = pl.kernel(
        _sc_gather_body,
        out_type=jax.ShapeDtypeStruct((_BATCH, _D), jnp.float32),
        mesh=plsc.VectorSubcoreMesh(core_axis_name="c", subcore_axis_name="s"),
        scratch_types=[
            pltpu.VMEM((_B_PER_W,), jnp.int32),
            pltpu.VMEM((_B_PER_W, _D), jnp.float32),
            pltpu.SemaphoreType.DMA,
        ],
    )(features, tgt)

    out = pl.pallas_call(
        _fin_kernel,
        out_shape=jax.ShapeDtypeStruct((1, 1), jnp.float32),
    )(inputs, gathered, s128)
    return out[0, 0] / _BATCH


# speedup vs baseline: 1.1819x; 1.0825x over previous
import functools

import jax
import jax.numpy as jnp
from jax import lax
from jax.experimental import pallas as pl
import jax.experimental.pallas.tpu as pltpu
from jax.experimental.pallas import tpu_sc as plsc

_BATCH = 4096
_N = 100000
_D = 128
_TEMP = 0.05
_NT = 1024
_LOG2E = 1.4426950408889634

_SC_NC = 2
_SC_NS = 16
_SC_NW = _SC_NC * _SC_NS
_B_PER_W = _BATCH // _SC_NW


def _norm_kernel(x_ref, xs_ref):
    x = x_ref[...]
    nrm = jnp.sqrt(jnp.sum(x * x, axis=1, keepdims=True))
    scale = _LOG2E / (jnp.maximum(nrm, 1e-12) * _TEMP)
    xs_ref[...] = (x * scale).astype(jnp.float8_e4m3fn)


def _sum_kernel(xs_ref, f_ref, s_ref):
    i = pl.program_id(0)

    @pl.when(i == 0)
    def _init():
        s_ref[...] = jnp.zeros_like(s_ref)

    l2 = lax.dot_general(
        xs_ref[...], f_ref[...],
        (((1,), (1,)), ((), ())),
        preferred_element_type=jnp.float32,
    )
    e = jnp.exp2(l2.astype(jnp.bfloat16))
    e128 = ((e[:, 0:128] + e[:, 128:256]) + (e[:, 256:384] + e[:, 384:512])) + (
        (e[:, 512:640] + e[:, 640:768]) + (e[:, 768:896] + e[:, 896:1024]))
    s_ref[...] += e128.astype(jnp.float32)


def _sc_gather_body(table_hbm, idx_hbm, out_hbm, idx_v, rows_v, sem):
    wid = lax.axis_index("s") * _SC_NC + lax.axis_index("c")
    base = wid * _B_PER_W
    pltpu.sync_copy(idx_hbm.at[pl.ds(base, _B_PER_W)], idx_v)
    pltpu.async_copy(table_hbm.at[idx_v], rows_v, sem).wait()
    pltpu.sync_copy(rows_v, out_hbm.at[pl.ds(base, _B_PER_W)])


def _fin_kernel(x_ref, g_ref, s_ref, out_ref):
    x = x_ref[...]
    nrm = jnp.sqrt(jnp.sum(x * x, axis=1, keepdims=True))
    t = jnp.sum(x * g_ref[...], axis=1, keepdims=True) / (
        jnp.maximum(nrm, 1e-12) * _TEMP)
    s = jnp.sum(s_ref[...], axis=1, keepdims=True)
    out_ref[...] = jnp.sum(jnp.log(s) - t, axis=0, keepdims=True)


@functools.partial(jax.jit, static_argnames=())
def kernel(inputs, targets, features):
    n_pad = pl.cdiv(_N, _NT) * _NT
    f = jnp.pad(features, ((0, n_pad - _N), (0, 0))).astype(jnp.float8_e4m3fn)
    tgt = targets.astype(jnp.int32)

    xs = pl.pallas_call(
        _norm_kernel,
        out_shape=jax.ShapeDtypeStruct((_BATCH, _D), jnp.float8_e4m3fn),
    )(inputs)

    s128 = pl.pallas_call(
        _sum_kernel,
        grid=(n_pad // _NT,),
        in_specs=[
            pl.BlockSpec((_BATCH, _D), lambda i: (0, 0)),
            pl.BlockSpec((_NT, _D), lambda i: (i, 0)),
        ],
        out_specs=pl.BlockSpec((_BATCH, _D), lambda i: (0, 0)),
        out_shape=jax.ShapeDtypeStruct((_BATCH, _D), jnp.float32),
    )(xs, f)

    gathered = pl.kernel(
        _sc_gather_body,
        out_type=jax.ShapeDtypeStruct((_BATCH, _D), jnp.float32),
        mesh=plsc.VectorSubcoreMesh(core_axis_name="c", subcore_axis_name="s"),
        scratch_types=[
            pltpu.VMEM((_B_PER_W,), jnp.int32),
            pltpu.VMEM((_B_PER_W, _D), jnp.float32),
            pltpu.SemaphoreType.DMA,
        ],
    )(features, tgt)

    out = pl.pallas_call(
        _fin_kernel,
        out_shape=jax.ShapeDtypeStruct((1, 1), jnp.float32),
    )(inputs, gathered, s128)
    return out[0, 0] / _BATCH


# speedup vs baseline: 1.2432x; 1.0519x over previous
import functools

import jax
import jax.numpy as jnp
from jax import lax
from jax.experimental import pallas as pl
import jax.experimental.pallas.tpu as pltpu
from jax.experimental.pallas import tpu_sc as plsc

_BATCH = 4096
_N = 100000
_D = 128
_TEMP = 0.05
_NT = 2048
_LOG2E = 1.4426950408889634

_SC_NC = 2
_SC_NS = 16
_SC_NW = _SC_NC * _SC_NS
_B_PER_W = _BATCH // _SC_NW


def _norm_kernel(x_ref, xs_ref):
    x = x_ref[...]
    nrm = jnp.sqrt(jnp.sum(x * x, axis=1, keepdims=True))
    scale = _LOG2E / (jnp.maximum(nrm, 1e-12) * _TEMP)
    xs_ref[...] = (x * scale).astype(jnp.float8_e4m3fn)


def _sum_kernel(xs_ref, f_ref, s_ref):
    i = pl.program_id(0)

    @pl.when(i == 0)
    def _init():
        s_ref[...] = jnp.zeros_like(s_ref)

    l2 = lax.dot_general(
        xs_ref[...], f_ref[...],
        (((1,), (1,)), ((), ())),
        preferred_element_type=jnp.float32,
    )
    e = jnp.exp2(l2.astype(jnp.bfloat16))
    parts = [e[:, k * 128:(k + 1) * 128] for k in range(_NT // 128)]
    while len(parts) > 1:
        parts = [parts[j] + parts[j + 1] for j in range(0, len(parts), 2)]
    e128 = parts[0]
    s_ref[...] += e128.astype(jnp.float32)


def _sc_gather_body(table_hbm, idx_hbm, out_hbm, idx_v, rows_v, sem):
    wid = lax.axis_index("s") * _SC_NC + lax.axis_index("c")
    base = wid * _B_PER_W
    pltpu.sync_copy(idx_hbm.at[pl.ds(base, _B_PER_W)], idx_v)
    pltpu.async_copy(table_hbm.at[idx_v], rows_v, sem).wait()
    pltpu.sync_copy(rows_v, out_hbm.at[pl.ds(base, _B_PER_W)])


def _fin_kernel(x_ref, g_ref, s_ref, out_ref):
    x = x_ref[...]
    nrm = jnp.sqrt(jnp.sum(x * x, axis=1, keepdims=True))
    t = jnp.sum(x * g_ref[...], axis=1, keepdims=True) / (
        jnp.maximum(nrm, 1e-12) * _TEMP)
    s = jnp.sum(s_ref[...], axis=1, keepdims=True)
    out_ref[...] = jnp.sum(jnp.log(s) - t, axis=0, keepdims=True)


@functools.partial(jax.jit, static_argnames=())
def kernel(inputs, targets, features):
    n_pad = pl.cdiv(_N, _NT) * _NT
    f = jnp.pad(features, ((0, n_pad - _N), (0, 0))).astype(jnp.float8_e4m3fn)
    tgt = targets.astype(jnp.int32)

    xs = pl.pallas_call(
        _norm_kernel,
        out_shape=jax.ShapeDtypeStruct((_BATCH, _D), jnp.float8_e4m3fn),
    )(inputs)

    s128 = pl.pallas_call(
        _sum_kernel,
        grid=(n_pad // _NT,),
        in_specs=[
            pl.BlockSpec((_BATCH, _D), lambda i: (0, 0)),
            pl.BlockSpec((_NT, _D), lambda i: (i, 0)),
        ],
        out_specs=pl.BlockSpec((_BATCH, _D), lambda i: (0, 0)),
        out_shape=jax.ShapeDtypeStruct((_BATCH, _D), jnp.float32),
    )(xs, f)

    gathered = pl.kernel(
        _sc_gather_body,
        out_type=jax.ShapeDtypeStruct((_BATCH, _D), jnp.float32),
        mesh=plsc.VectorSubcoreMesh(core_axis_name="c", subcore_axis_name="s"),
        scratch_types=[
            pltpu.VMEM((_B_PER_W,), jnp.int32),
            pltpu.VMEM((_B_PER_W, _D), jnp.float32),
            pltpu.SemaphoreType.DMA,
        ],
    )(features, tgt)

    out = pl.pallas_call(
        _fin_kernel,
        out_shape=jax.ShapeDtypeStruct((1, 1), jnp.float32),
    )(inputs, gathered, s128)
    return out[0, 0] / _BATCH


# speedup vs baseline: 1.3880x; 1.1164x over previous
import functools

import jax
import jax.numpy as jnp
from jax import lax
from jax.experimental import pallas as pl
import jax.experimental.pallas.tpu as pltpu
from jax.experimental.pallas import tpu_sc as plsc

_BATCH = 4096
_N = 100000
_D = 128
_TEMP = 0.05
_NT = 2048
_LOG2E = 1.4426950408889634

_SC_NC = 2
_SC_NS = 16
_SC_NW = _SC_NC * _SC_NS
_B_PER_W = _BATCH // _SC_NW


def _norm_kernel(x_ref, xs_ref):
    x = x_ref[...]
    nrm = jnp.sqrt(jnp.sum(x * x, axis=1, keepdims=True))
    scale = _LOG2E / (jnp.maximum(nrm, 1e-12) * _TEMP)
    xs_ref[...] = (x * scale).astype(jnp.float8_e4m3fn)


def _sum_kernel(xs_ref, f_ref, s_ref):
    i = pl.program_id(0)

    @pl.when(i == 0)
    def _init():
        s_ref[...] = jnp.zeros_like(s_ref)

    row = i * _NT + lax.broadcasted_iota(jnp.int32, (_NT, 1), 0)
    ft = jnp.where(row < _N, f_ref[...], 0.0).astype(jnp.float8_e4m3fn)

    l2 = lax.dot_general(
        xs_ref[...], ft,
        (((1,), (1,)), ((), ())),
        preferred_element_type=jnp.float32,
    )
    e = jnp.exp2(l2.astype(jnp.bfloat16))
    parts = [e[:, k * 128:(k + 1) * 128] for k in range(_NT // 128)]
    while len(parts) > 1:
        parts = [parts[j] + parts[j + 1] for j in range(0, len(parts), 2)]
    s_ref[...] += parts[0].astype(jnp.float32)


def _sc_gather_body(table_hbm, idx_hbm, out_hbm, idx_v, rows_v, sem):
    wid = lax.axis_index("s") * _SC_NC + lax.axis_index("c")
    base = wid * _B_PER_W
    pltpu.sync_copy(idx_hbm.at[pl.ds(base, _B_PER_W)], idx_v)
    pltpu.async_copy(table_hbm.at[idx_v], rows_v, sem).wait()
    pltpu.sync_copy(rows_v, out_hbm.at[pl.ds(base, _B_PER_W)])


def _fin_kernel(x_ref, g_ref, s_ref, out_ref):
    n_pad_rows = pl.cdiv(_N, _NT) * _NT - _N
    x = x_ref[...]
    nrm = jnp.sqrt(jnp.sum(x * x, axis=1, keepdims=True))
    t = jnp.sum(x * g_ref[...], axis=1, keepdims=True) / (
        jnp.maximum(nrm, 1e-12) * _TEMP)
    s = jnp.sum(s_ref[...], axis=1, keepdims=True) - float(n_pad_rows)
    out_ref[...] = jnp.sum(jnp.log(s) - t, axis=0, keepdims=True)


@functools.partial(jax.jit, static_argnames=())
def kernel(inputs, targets, features):
    tgt = targets.astype(jnp.int32)

    xs = pl.pallas_call(
        _norm_kernel,
        out_shape=jax.ShapeDtypeStruct((_BATCH, _D), jnp.float8_e4m3fn),
    )(inputs)

    s128 = pl.pallas_call(
        _sum_kernel,
        grid=(pl.cdiv(_N, _NT),),
        in_specs=[
            pl.BlockSpec((_BATCH, _D), lambda i: (0, 0)),
            pl.BlockSpec((_NT, _D), lambda i: (i, 0)),
        ],
        out_specs=pl.BlockSpec((_BATCH, _D), lambda i: (0, 0)),
        out_shape=jax.ShapeDtypeStruct((_BATCH, _D), jnp.float32),
    )(xs, features)

    gathered = pl.kernel(
        _sc_gather_body,
        out_type=jax.ShapeDtypeStruct((_BATCH, _D), jnp.float32),
        mesh=plsc.VectorSubcoreMesh(core_axis_name="c", subcore_axis_name="s"),
        scratch_types=[
            pltpu.VMEM((_B_PER_W,), jnp.int32),
            pltpu.VMEM((_B_PER_W, _D), jnp.float32),
            pltpu.SemaphoreType.DMA,
        ],
    )(features, tgt)

    out = pl.pallas_call(
        _fin_kernel,
        out_shape=jax.ShapeDtypeStruct((1, 1), jnp.float32),
    )(inputs, gathered, s128)
    return out[0, 0] / _BATCH
